# baseline (speedup 1.0000x reference)
import functools

import jax
import jax.numpy as jnp
from jax import lax
from jax.experimental import pallas as pl
from jax.experimental.pallas import tpu as pltpu
from jax.experimental.pallas import tpu_sc as plsc

_NC = 2
_NS = 16
_NW = _NC * _NS

_CHUNK = 128
_NBUF = 4


@functools.partial(jax.jit, static_argnames=("n_chunks", "d"))
def _emb_lookup(idx, weight, *, n_chunks, d):
  out_shape = jax.ShapeDtypeStruct((_NW, n_chunks, _CHUNK, d), jnp.float32)

  @functools.partial(
      pl.kernel,
      out_type=out_shape,
      mesh=plsc.VectorSubcoreMesh(core_axis_name="c", subcore_axis_name="s"),
      scratch_types=[
          pltpu.VMEM((n_chunks, _CHUNK), jnp.int32),
          pltpu.VMEM((_NBUF, _CHUNK, d), jnp.float32),
          [pltpu.SemaphoreType.DMA] * _NBUF,
          [pltpu.SemaphoreType.DMA] * _NBUF,
      ],
      compiler_params=pltpu.CompilerParams(use_tc_tiling_on_sc=False),
  )
  def k(table_hbm, idx_hbm, out_hbm, idx_v, rows_v, gsems, ssems):
    w = lax.axis_index("s") * _NC + lax.axis_index("c")
    pltpu.sync_copy(idx_hbm.at[w], idx_v)

    for b in range(_NBUF):
      pltpu.async_copy(table_hbm.at[idx_v.at[b]], rows_v.at[b], gsems[b])

    @pl.loop(0, n_chunks, step=_NBUF)
    def _(g):
      for b in range(_NBUF):
        c = g + b
        pltpu.make_async_copy(out_hbm.at[0, 0], rows_v.at[b], gsems[b]).wait()
        pltpu.async_copy(rows_v.at[b], out_hbm.at[w, c], ssems[b])
        nc = c + _NBUF

        @pl.when(nc < n_chunks)
        def _():
          pltpu.make_async_copy(rows_v.at[b], out_hbm.at[0, 0], ssems[b]).wait()
          pltpu.async_copy(table_hbm.at[idx_v.at[nc]], rows_v.at[b], gsems[b])

    for b in range(_NBUF):
      pltpu.make_async_copy(rows_v.at[b], out_hbm.at[0, 0], ssems[b]).wait()

  return k(weight, idx)


def kernel(x, weight):
  d = weight.shape[-1]
  n = x.size
  n_chunks = n // (_NW * _CHUNK)
  idx = x.reshape(_NW, n_chunks, _CHUNK).astype(jnp.int32)
  out = _emb_lookup(idx, weight, n_chunks=n_chunks, d=d)
  return out.reshape(x.shape + (d,))
